# Initial kernel scaffold; baseline (speedup 1.0000x reference)
import jax
import jax.numpy as jnp
from jax.experimental import pallas as pl


def kernel(distance_matrices_batch, num_species_batch):
    raise NotImplementedError("write your pallas kernel here")


# speedup vs baseline: 1320.1282x; 1320.1282x over previous
import math

import jax
import jax.numpy as jnp
from jax import lax
from jax.experimental import pallas as pl
from jax.experimental.pallas import tpu as pltpu
from jax.experimental.pallas import tpu_sc as plsc

RCR = 5.2
NEG_ETA = -16.0
NUM_SPECIES = 4
P = 16
L = 16
B, A = 16, 256
ROWS = B * A
NC, NS = 2, 16
NW = NC * NS
RPW = ROWS // NW
CHUNKS = A // L
OUTW = NUM_SPECIES * P

_FC_COEFS = tuple(0.5 * (-1.0) ** k / math.factorial(2 * k) for k in range(8))
_FC_COEFS = (_FC_COEFS[0] + 0.5,) + _FC_COEFS[1:]
_XSCALE = math.pi / RCR


def _fc_poly(dv):
    x = dv * _XSCALE
    u = x * x
    acc = jnp.full((L,), _FC_COEFS[-1], dtype=jnp.float32)
    for c in reversed(_FC_COEFS[:-1]):
        acc = acc * u + c
    return acc


def _sc_body(d_hbm, s_hbm, out_hbm, drows, srow16, dbuf, fbuf, sbuf, obuf):
    cid = lax.axis_index("c")
    sid = lax.axis_index("s")
    w = sid * NC + cid
    row0 = w * RPW
    bt = row0 // A

    pltpu.sync_copy(d_hbm.at[pl.ds(row0 * A, RPW * A)], drows)
    pltpu.sync_copy(s_hbm.at[pl.ds(bt * A, A)], srow16)

    iota = lax.iota(jnp.int32, L)
    iota_f = iota.astype(jnp.float32)
    shfr = 0.9 + 0.26875 * iota_f
    zf = jnp.zeros((L,), jnp.float32)
    zi = jnp.zeros((L,), jnp.int32)

    for c in range(CHUNKS):
        sv = srow16[pl.ds(c * L, L)]
        srow16[pl.ds(c * L, L)] = (sv - 1) * P

    def row_body(row, _):
        rowoff = row * A
        rowbase = row * OUTW

        for q in range(OUTW // L):
            plsc.store_scatter(obuf, [rowbase + q * L + iota], zf)

        cnt = jnp.zeros((L,), jnp.int32)
        for c in range(CHUNKS):
            dv = plsc.load_gather(drows, [rowoff + c * L + iota])
            valid = (dv != 0.0) & (dv < RCR)
            fc = _fc_poly(dv)
            sv = srow16[pl.ds(c * L, L)]
            pos = cnt + plsc.cumsum(valid.astype(jnp.int32)) - 1
            plsc.store_scatter(dbuf, [pos], dv, mask=valid)
            plsc.store_scatter(fbuf, [pos], fc, mask=valid)
            plsc.store_scatter(sbuf, [pos], sv, mask=valid)
            cnt = cnt + plsc.all_reduce_population_count(valid)

        pad = cnt + iota
        plsc.store_scatter(dbuf, [pad], zf)
        plsc.store_scatter(fbuf, [pad], zf)
        plsc.store_scatter(sbuf, [pad], zi)

        n = cnt[0]
        niter = (n + (L - 1)) // L

        def group_body(g, carry):
            base = pl.multiple_of(g * L, L)
            dv16 = dbuf[pl.ds(base, L)]
            fv16 = fbuf[pl.ds(base, L)]
            sv16 = sbuf[pl.ds(base, L)]
            for u in range(L):
                t = dv16[u] - shfr
                gv = jnp.exp((t * t) * NEG_ETA) * fv16[u]
                plsc.addupdate_scatter(obuf, [(rowbase + sv16[u]) + iota], gv)
            return carry

        lax.fori_loop(0, niter, group_body, 0)
        return _

    lax.fori_loop(0, RPW, row_body, 0)

    pltpu.sync_copy(obuf, out_hbm.at[pl.ds(row0 * OUTW, RPW * OUTW)])


def _make_sc_call():
    mesh = plsc.VectorSubcoreMesh(
        core_axis_name="c", subcore_axis_name="s", num_cores=NC, num_subcores=NS
    )
    return pl.kernel(
        _sc_body,
        out_type=jax.ShapeDtypeStruct((ROWS * OUTW,), jnp.float32),
        mesh=mesh,
        compiler_params=pltpu.CompilerParams(needs_layout_passes=False),
        scratch_types=[
            pltpu.VMEM((RPW * A,), jnp.float32),
            pltpu.VMEM((A,), jnp.int32),
            pltpu.VMEM((A + L,), jnp.float32),
            pltpu.VMEM((A + L,), jnp.float32),
            pltpu.VMEM((A + L,), jnp.int32),
            pltpu.VMEM((RPW * OUTW,), jnp.float32),
        ],
    )


def kernel(distance_matrices_batch, num_species_batch):
    d = distance_matrices_batch.reshape(ROWS * A)
    s = num_species_batch.astype(jnp.int32).reshape(B * A)
    out = _make_sc_call()(d, s)
    return out.reshape(B, A, OUTW)
